# initial kernel scaffold (unmeasured)
import jax
import jax.numpy as jnp
from jax import lax
from jax.experimental import pallas as pl
from jax.experimental.pallas import tpu as pltpu


def kernel(
    x,
):
    def body(*refs):
        pass

    out_shape = jax.ShapeDtypeStruct(..., jnp.float32)
    return pl.pallas_call(body, out_shape=out_shape)(...)


# baseline (device time: 110499 ns/iter reference)
import jax
import jax.numpy as jnp
from jax import lax
from jax.experimental import pallas as pl
from jax.experimental.pallas import tpu as pltpu

N_DEV = 32


def _bitonic_merge_stages(xs, m_total, n, k_start):
    row = lax.broadcasted_iota(jnp.int32, (m_total, n), 0)
    k = k_start
    while k <= m_total:
        j = k // 2
        while j >= 1:
            if j >= 8:
                nb = m_total // (2 * j)
                v = xs.reshape(nb, 2, j, n)
                lo = v[:, 0]
                hi = v[:, 1]
                mn = jnp.minimum(lo, hi)
                mx = jnp.maximum(lo, hi)
                q = k // (2 * j)
                b = lax.broadcasted_iota(jnp.int32, (nb, j, n), 0)
                asc = (b // q) % 2 == 0
                new = jnp.stack(
                    [jnp.where(asc, mn, mx), jnp.where(asc, mx, mn)], axis=1
                )
                xs = new.reshape(m_total, n)
            else:
                up = jnp.roll(xs, -j, axis=0)
                dn = jnp.roll(xs, j, axis=0)
                bit = (row // j) % 2
                p = jnp.where(bit == 0, up, dn)
                asc = ((row // k) % 2) == 0
                take_min = (bit == 0) == asc
                xs = jnp.where(
                    take_min, jnp.minimum(xs, p), jnp.maximum(xs, p)
                )
            j //= 2
        k *= 2
    return xs


def kernel(x):
    m_per, n = x.shape
    m_total = N_DEV * m_per

    my = lax.axis_index("i")
    s = jnp.sort(x, axis=0)
    s = jnp.where((my % 2) == 0, s, s[::-1])

    def body(x_ref, out_ref, gather_ref, send_sems, recv_sems):
        my_pos = lax.axis_index("i")
        left = lax.rem(my_pos + N_DEV - 1, N_DEV)
        right = lax.rem(my_pos + 1, N_DEV)

        barrier_sem = pltpu.get_barrier_semaphore()
        for nbr in (left, right):
            pl.semaphore_signal(
                barrier_sem,
                inc=1,
                device_id=(nbr,),
                device_id_type=pl.DeviceIdType.MESH,
            )
        pl.semaphore_wait(barrier_sem, 2)

        gather_ref[my_pos] = x_ref[...]

        for h in range(N_DEV - 1):
            src_pos = lax.rem(my_pos - h + N_DEV, N_DEV)
            rdma = pltpu.make_async_remote_copy(
                src_ref=gather_ref.at[src_pos],
                dst_ref=gather_ref.at[src_pos],
                send_sem=send_sems.at[h],
                recv_sem=recv_sems.at[h],
                device_id=(right,),
                device_id_type=pl.DeviceIdType.MESH,
            )
            rdma.start()
            rdma.wait()

        xs = gather_ref[...].reshape(m_total, n)
        xs = _bitonic_merge_stages(xs, m_total, n, 2 * m_per)

        gather_ref[...] = xs.reshape(N_DEV, m_per, n)
        out_ref[...] = gather_ref[my_pos]

    return pl.pallas_call(
        body,
        out_shape=jax.ShapeDtypeStruct((m_per, n), jnp.float32),
        in_specs=[pl.BlockSpec(memory_space=pltpu.VMEM)],
        out_specs=pl.BlockSpec(memory_space=pltpu.VMEM),
        scratch_shapes=[
            pltpu.VMEM((N_DEV, m_per, n), jnp.float32),
            pltpu.SemaphoreType.DMA((N_DEV - 1,)),
            pltpu.SemaphoreType.DMA((N_DEV - 1,)),
        ],
        compiler_params=pltpu.CompilerParams(collective_id=0),
    )(s)


# device time: 64669 ns/iter; 1.7087x vs baseline; 1.7087x over previous
import jax
import jax.numpy as jnp
from jax import lax
from jax.experimental import pallas as pl
from jax.experimental.pallas import tpu as pltpu

N_DEV = 32
N_STEPS = 5


def _merge_levels(xs, m2, n, asc):
    row = None
    j = m2 // 2
    while j >= 1:
        if j >= 8:
            nb = m2 // (2 * j)
            v = xs.reshape(nb, 2, j, n)
            lo = v[:, 0]
            hi = v[:, 1]
            mn = jnp.minimum(lo, hi)
            mx = jnp.maximum(lo, hi)
            new = jnp.stack(
                [jnp.where(asc, mn, mx), jnp.where(asc, mx, mn)], axis=1
            )
            xs = new.reshape(m2, n)
        else:
            if row is None:
                row = lax.broadcasted_iota(jnp.int32, (m2, n), 0)
            up = jnp.roll(xs, -j, axis=0)
            dn = jnp.roll(xs, j, axis=0)
            bit = (row // j) % 2
            p = jnp.where(bit == 0, up, dn)
            take_min = (bit == 0) == asc
            xs = jnp.where(take_min, jnp.minimum(xs, p), jnp.maximum(xs, p))
        j //= 2
    return xs


def kernel(x):
    m_per, n = x.shape

    my = lax.axis_index("i")
    s = jnp.sort(x, axis=0)
    s = jnp.where((my % 2) == 0, s, s[::-1])

    def body(x_ref, out_ref, gather_ref, send_sems, recv_sems):
        my_pos = lax.axis_index("i")

        barrier_sem = pltpu.get_barrier_semaphore()
        for t in range(N_STEPS):
            pl.semaphore_signal(
                barrier_sem,
                inc=1,
                device_id=(my_pos ^ (1 << t),),
                device_id_type=pl.DeviceIdType.MESH,
            )
        pl.semaphore_wait(barrier_sem, N_STEPS)

        gather_ref[my_pos] = x_ref[...]

        for step in range(N_STEPS):
            sz = 1 << step
            partner = my_pos ^ sz
            start = (my_pos // sz) * sz
            rdma = pltpu.make_async_remote_copy(
                src_ref=gather_ref.at[pl.ds(start, sz)],
                dst_ref=gather_ref.at[pl.ds(start, sz)],
                send_sem=send_sems.at[step],
                recv_sem=recv_sems.at[step],
                device_id=(partner,),
                device_id_type=pl.DeviceIdType.MESH,
            )
            rdma.start()
            rdma.wait()

            grp = my_pos // (2 * sz)
            start2 = grp * (2 * sz)
            m2 = 2 * sz * m_per
            v = gather_ref[pl.ds(start2, 2 * sz)].reshape(m2, n)
            if step == N_STEPS - 1:
                v = _merge_levels(v, m2, n, True)
            else:
                v = _merge_levels(v, m2, n, (grp % 2) == 0)
            gather_ref[pl.ds(start2, 2 * sz)] = v.reshape(2 * sz, m_per, n)

        out_ref[...] = gather_ref[my_pos]

    return pl.pallas_call(
        body,
        out_shape=jax.ShapeDtypeStruct((m_per, n), jnp.float32),
        in_specs=[pl.BlockSpec(memory_space=pltpu.VMEM)],
        out_specs=pl.BlockSpec(memory_space=pltpu.VMEM),
        scratch_shapes=[
            pltpu.VMEM((N_DEV, m_per, n), jnp.float32),
            pltpu.SemaphoreType.DMA((N_STEPS,)),
            pltpu.SemaphoreType.DMA((N_STEPS,)),
        ],
        compiler_params=pltpu.CompilerParams(collective_id=0),
    )(s)


# device time: 54119 ns/iter; 2.0418x vs baseline; 1.1949x over previous
import jax
import jax.numpy as jnp
from jax import lax
from jax.experimental import pallas as pl
from jax.experimental.pallas import tpu as pltpu

N_DEV = 32
N_STEPS = 5


def _coords_of_mesh(i):
    z = i // 8
    p = i % 8
    y = p // 2
    x = (p % 2) ^ (y & 1)
    return x, y, z


def _slot_of_mesh(i):
    x, y, z = _coords_of_mesh(i)
    return x * 16 + (y & 1) * 8 + (z & 1) * 4 + (y // 2) * 2 + (z // 2)


def _mesh_of_slot(s):
    x = (s // 16) & 1
    y = ((s // 2) & 1) * 2 + ((s // 8) & 1)
    z = (s & 1) * 2 + ((s // 4) & 1)
    p = 2 * y + (x ^ (y & 1))
    return 8 * z + p


def _merge_levels(xs, m2, n, asc):
    row = None
    j = m2 // 2
    while j >= 1:
        if j >= 8:
            nb = m2 // (2 * j)
            v = xs.reshape(nb, 2, j, n)
            lo = v[:, 0]
            hi = v[:, 1]
            mn = jnp.minimum(lo, hi)
            mx = jnp.maximum(lo, hi)
            new = jnp.stack(
                [jnp.where(asc, mn, mx), jnp.where(asc, mx, mn)], axis=1
            )
            xs = new.reshape(m2, n)
        else:
            if row is None:
                row = lax.broadcasted_iota(jnp.int32, (m2, n), 0)
            up = jnp.roll(xs, -j, axis=0)
            dn = jnp.roll(xs, j, axis=0)
            bit = (row // j) % 2
            p = jnp.where(bit == 0, up, dn)
            take_min = (bit == 0) == asc
            xs = jnp.where(take_min, jnp.minimum(xs, p), jnp.maximum(xs, p))
        j //= 2
    return xs


def kernel(x):
    m_per, n = x.shape

    my = lax.axis_index("i")
    my_slot_out = _slot_of_mesh(my)
    s = jnp.sort(x, axis=0)
    s = jnp.where((my_slot_out % 2) == 0, s, s[::-1])

    def body(x_ref, out_ref, gather_ref, send_sems, recv_sems):
        my_pos = lax.axis_index("i")
        my_slot = _slot_of_mesh(my_pos)

        barrier_sem = pltpu.get_barrier_semaphore()
        for t in range(N_STEPS):
            pl.semaphore_signal(
                barrier_sem,
                inc=1,
                device_id=(_mesh_of_slot(my_slot ^ (1 << t)),),
                device_id_type=pl.DeviceIdType.MESH,
            )
        pl.semaphore_wait(barrier_sem, N_STEPS)

        gather_ref[my_slot] = x_ref[...]

        for step in range(N_STEPS):
            sz = 1 << step
            partner = _mesh_of_slot(my_slot ^ sz)
            start = (my_slot // sz) * sz
            rdma = pltpu.make_async_remote_copy(
                src_ref=gather_ref.at[pl.ds(start, sz)],
                dst_ref=gather_ref.at[pl.ds(start, sz)],
                send_sem=send_sems.at[step],
                recv_sem=recv_sems.at[step],
                device_id=(partner,),
                device_id_type=pl.DeviceIdType.MESH,
            )
            rdma.start()
            rdma.wait_recv()

            grp = my_slot // (2 * sz)
            start2 = grp * (2 * sz)
            m2 = 2 * sz * m_per
            v = gather_ref[pl.ds(start2, 2 * sz)].reshape(m2, n)
            if step == N_STEPS - 1:
                v = _merge_levels(v, m2, n, True)
            else:
                v = _merge_levels(v, m2, n, (grp % 2) == 0)
            rdma.wait_send()
            gather_ref[pl.ds(start2, 2 * sz)] = v.reshape(2 * sz, m_per, n)

        out_ref[...] = gather_ref[my_pos]

    return pl.pallas_call(
        body,
        out_shape=jax.ShapeDtypeStruct((m_per, n), jnp.float32),
        in_specs=[pl.BlockSpec(memory_space=pltpu.VMEM)],
        out_specs=pl.BlockSpec(memory_space=pltpu.VMEM),
        scratch_shapes=[
            pltpu.VMEM((N_DEV, m_per, n), jnp.float32),
            pltpu.SemaphoreType.DMA((N_STEPS,)),
            pltpu.SemaphoreType.DMA((N_STEPS,)),
        ],
        compiler_params=pltpu.CompilerParams(collective_id=0),
    )(s)


# device time: 51564 ns/iter; 2.1429x vs baseline; 1.0496x over previous
import functools

import jax
import jax.numpy as jnp
from jax import lax
from jax.experimental import pallas as pl
from jax.experimental.pallas import tpu as pltpu

N_DEV = 32
N_STEPS = 5


def _coords_of_mesh(i):
    z = i // 8
    p = i % 8
    y = p // 2
    x = (p % 2) ^ (y & 1)
    return x, y, z


def _slot_of_mesh(i):
    x, y, z = _coords_of_mesh(i)
    return x * 16 + (y & 1) * 8 + (z & 1) * 4 + (y // 2) * 2 + (z // 2)


def _mesh_of_slot(s):
    x = (s // 16) & 1
    y = ((s // 2) & 1) * 2 + ((s // 8) & 1)
    z = (s & 1) * 2 + ((s // 4) & 1)
    p = 2 * y + (x ^ (y & 1))
    return 8 * z + p


def _merge_levels(xs, m2, n, asc):
    row = None
    j = m2 // 2
    while j >= 1:
        if j >= 8:
            nb = m2 // (2 * j)
            v = xs.reshape(nb, 2, j, n)
            lo = v[:, 0]
            hi = v[:, 1]
            mn = jnp.minimum(lo, hi)
            mx = jnp.maximum(lo, hi)
            new = jnp.stack(
                [jnp.where(asc, mn, mx), jnp.where(asc, mx, mn)], axis=1
            )
            xs = new.reshape(m2, n)
        else:
            if row is None:
                row = lax.broadcasted_iota(jnp.int32, (m2, n), 0)
            up = jnp.roll(xs, -j, axis=0)
            dn = jnp.roll(xs, j, axis=0)
            bit = (row // j) % 2
            p = jnp.where(bit == 0, up, dn)
            take_min = (bit == 0) == asc
            xs = jnp.where(take_min, jnp.minimum(xs, p), jnp.maximum(xs, p))
        j //= 2
    return xs


def kernel(x):
    m_per, n = x.shape

    my = lax.axis_index("i")
    my_slot_out = _slot_of_mesh(my)
    s = jnp.sort(x, axis=0)
    s = jnp.where((my_slot_out % 2) == 0, s, s[::-1])

    def body(x_ref, out_ref, buf_ref, send_sems, recv_sems):
        my_pos = lax.axis_index("i")
        my_slot = _slot_of_mesh(my_pos)
        partners = [_mesh_of_slot(my_slot ^ (1 << t)) for t in range(N_STEPS)]

        barrier_sem = pltpu.get_barrier_semaphore()
        for nbr in partners:
            pl.semaphore_signal(
                barrier_sem,
                inc=1,
                device_id=(nbr,),
                device_id_type=pl.DeviceIdType.MESH,
            )
        pl.semaphore_wait(barrier_sem, N_STEPS)

        buf_ref[0] = x_ref[...]

        for step in range(N_STEPS):
            sz = 1 << step
            rdma = pltpu.make_async_remote_copy(
                src_ref=buf_ref.at[pl.ds(0, sz)],
                dst_ref=buf_ref.at[pl.ds(sz, sz)],
                send_sem=send_sems.at[step],
                recv_sem=recv_sems.at[step],
                device_id=(partners[step],),
                device_id_type=pl.DeviceIdType.MESH,
            )
            rdma.start()
            rdma.wait()

            m_blk = sz * m_per
            mine = buf_ref[pl.ds(0, sz)].reshape(m_blk, n)
            theirs = buf_ref[pl.ds(sz, sz)].reshape(m_blk, n)
            is_low = ((my_slot // sz) % 2) == 0
            v = jnp.where(
                is_low,
                jnp.concatenate([mine, theirs], axis=0),
                jnp.concatenate([theirs, mine], axis=0),
            )
            grp = my_slot // (2 * sz)
            if step == N_STEPS - 1:
                v = _merge_levels(v, 2 * m_blk, n, True)
            else:
                v = _merge_levels(v, 2 * m_blk, n, (grp % 2) == 0)
            buf_ref[pl.ds(0, 2 * sz)] = v.reshape(2 * sz, m_per, n)

        out_ref[...] = buf_ref[my_pos]

        @functools.partial(
            pl.run_scoped, second_barrier=pltpu.SemaphoreType.REGULAR
        )
        def _(second_barrier):
            for nbr in partners:
                pl.semaphore_signal(
                    second_barrier,
                    inc=1,
                    device_id=(nbr,),
                    device_id_type=pl.DeviceIdType.MESH,
                )
            pl.semaphore_wait(second_barrier, N_STEPS)

    return pl.pallas_call(
        body,
        out_shape=jax.ShapeDtypeStruct((m_per, n), jnp.float32),
        in_specs=[pl.BlockSpec(memory_space=pltpu.VMEM)],
        out_specs=pl.BlockSpec(memory_space=pltpu.VMEM),
        scratch_shapes=[
            pltpu.VMEM((N_DEV, m_per, n), jnp.float32),
            pltpu.SemaphoreType.DMA((N_STEPS,)),
            pltpu.SemaphoreType.DMA((N_STEPS,)),
        ],
        compiler_params=pltpu.CompilerParams(collective_id=7),
    )(s)
